# baseline (speedup 1.0000x reference)
import jax
import jax.numpy as jnp
from jax import lax
from jax.experimental import pallas as pl
from jax.experimental.pallas import tpu as pltpu
from jax.experimental.pallas import tpu_sc as plsc

E = 64
N = 32768
L = 16
NC, NS = 2, 16
NW = NC * NS
ROWS_W = N // NW
CHUNK = 128
NSTEPS = ROWS_W // CHUNK
GROUPS = CHUNK // L


def _gating_body(in_hbm, out_hbm, in_v, out_v, best_v):
    wid = lax.axis_index("s") * NC + lax.axis_index("c")
    lanes = lax.iota(jnp.int32, L)
    zero_f = jnp.zeros((L,), jnp.float32)
    one_f = jnp.ones((L,), jnp.float32)

    @pl.loop(0, CHUNK * E // L)
    def _zero(i):
        out_v[pl.ds(i * L, L)] = zero_f

    base_w = wid * ROWS_W

    @pl.loop(0, NSTEPS)
    def _step(s):
        elem0 = (base_w + s * CHUNK) * E
        pltpu.sync_copy(in_hbm.at[pl.ds(elem0, CHUNK * E)], in_v)

        @pl.loop(0, GROUPS)
        def _group(g):
            row_base = (g * L + lanes) * E
            m = plsc.load_gather(in_v, [row_base])
            best = jnp.zeros((L,), jnp.int32)
            for c in range(1, E):
                v = plsc.load_gather(in_v, [row_base + c])
                gt = v > m
                m = jnp.where(gt, v, m)
                best = jnp.where(gt, jnp.full((L,), c, jnp.int32), best)
            plsc.store_scatter(out_v, [row_base + best], one_f)
            best_v[pl.ds(g * L, L)] = best

        pltpu.sync_copy(out_v, out_hbm.at[pl.ds(elem0, CHUNK * E)])

        @pl.loop(0, GROUPS)
        def _rezero(g):
            row_base = (g * L + lanes) * E
            best = best_v[pl.ds(g * L, L)]
            plsc.store_scatter(out_v, [row_base + best], zero_f)


def kernel(snr_probs):
    k = pl.kernel(
        _gating_body,
        out_type=jax.ShapeDtypeStruct((N * E,), jnp.float32),
        mesh=plsc.VectorSubcoreMesh(core_axis_name="c", subcore_axis_name="s"),
        compiler_params=pltpu.CompilerParams(needs_layout_passes=False),
        scratch_types=[
            pltpu.VMEM((CHUNK * E,), jnp.float32),
            pltpu.VMEM((CHUNK * E,), jnp.float32),
            pltpu.VMEM((CHUNK,), jnp.int32),
        ],
    )
    return k(snr_probs.reshape(N * E)).reshape(N, E)


# speedup vs baseline: 1.0178x; 1.0178x over previous
import jax
import jax.numpy as jnp
from jax import lax
from jax.experimental import pallas as pl
from jax.experimental.pallas import tpu as pltpu
from jax.experimental.pallas import tpu_sc as plsc

E = 64
N = 32768
L = 16
NC, NS = 2, 16
NW = NC * NS
ROWS_W = N // NW
CHUNK = 256
NSTEPS = ROWS_W // CHUNK
GROUPS = CHUNK // L
NBUF = 2
CE = CHUNK * E


def _gating_body(in_hbm, out_hbm, in_v, out_v, best_v,
                 sem_in0, sem_in1, sem_out0, sem_out1):
    wid = lax.axis_index("s") * NC + lax.axis_index("c")
    lanes = lax.iota(jnp.int32, L)
    zero_f = jnp.zeros((L,), jnp.float32)
    one_f = jnp.ones((L,), jnp.float32)
    zero_i = jnp.zeros((L,), jnp.int32)
    sems_in = (sem_in0, sem_in1)
    sems_out = (sem_out0, sem_out1)

    @pl.loop(0, NBUF * CE // L)
    def _zero(i):
        out_v[pl.ds(i * L, L)] = zero_f

    @pl.loop(0, NBUF * CHUNK // L)
    def _zero_b(i):
        best_v[pl.ds(i * L, L)] = zero_i

    base_e = wid * ROWS_W * E

    def in_copy(s, b):
        return pltpu.make_async_copy(
            in_hbm.at[pl.ds(base_e + s * CE, CE)],
            in_v.at[pl.ds(b * CE, CE)],
            sems_in[b])

    def out_copy(s, b):
        return pltpu.make_async_copy(
            out_v.at[pl.ds(b * CE, CE)],
            out_hbm.at[pl.ds(base_e + s * CE, CE)],
            sems_out[b])

    in_copy(0, 0).start()
    in_copy(1, 1).start()

    for s in range(NSTEPS):
        b = s % NBUF
        in_copy(s, b).wait()
        if s >= NBUF:
            out_copy(s - NBUF, b).wait()

        @pl.loop(0, GROUPS)
        def _group(g, _s=s, _b=b):
            row_e = (g * L + lanes) * E
            ibase = row_e + _b * CE
            obase = row_e + _b * CE
            oldb = best_v[pl.ds(_b * CHUNK + g * L, L)]
            plsc.store_scatter(out_v, [obase + oldb], zero_f)
            m = plsc.load_gather(in_v, [ibase])
            best = zero_i
            for c in range(1, E):
                v = plsc.load_gather(in_v, [ibase + c])
                gt = v > m
                m = jnp.maximum(m, v)
                best = jnp.where(gt, jnp.full((L,), c, jnp.int32), best)
            plsc.store_scatter(out_v, [obase + best], one_f)
            best_v[pl.ds(_b * CHUNK + g * L, L)] = best

        out_copy(s, b).start()
        if s + NBUF < NSTEPS:
            in_copy(s + NBUF, b).start()

    out_copy(NSTEPS - 2, (NSTEPS - 2) % NBUF).wait()
    out_copy(NSTEPS - 1, (NSTEPS - 1) % NBUF).wait()


def kernel(snr_probs):
    k = pl.kernel(
        _gating_body,
        out_type=jax.ShapeDtypeStruct((N * E,), jnp.float32),
        mesh=plsc.VectorSubcoreMesh(core_axis_name="c", subcore_axis_name="s"),
        compiler_params=pltpu.CompilerParams(needs_layout_passes=False),
        scratch_types=[
            pltpu.VMEM((NBUF * CE,), jnp.float32),
            pltpu.VMEM((NBUF * CE,), jnp.float32),
            pltpu.VMEM((NBUF * CHUNK,), jnp.int32),
            pltpu.SemaphoreType.DMA,
            pltpu.SemaphoreType.DMA,
            pltpu.SemaphoreType.DMA,
            pltpu.SemaphoreType.DMA,
        ],
    )
    return k(snr_probs.reshape(N * E)).reshape(N, E)


# speedup vs baseline: 1.7412x; 1.7107x over previous
import jax
import jax.numpy as jnp
from jax import lax
from jax.experimental import pallas as pl
from jax.experimental.pallas import tpu as pltpu
from jax.experimental.pallas import tpu_sc as plsc

E = 64
N = 32768
L = 16
NC, NS = 2, 16
NW = NC * NS
ROWS_W = N // NW
CHUNK = 128
NSTEPS = ROWS_W // CHUNK
GROUPS = CHUNK // L
NBUF = 2


def _gating_body(in_hbm, out_hbm,
                 in0, in1, out0, out1, best0, best1,
                 sem_in0, sem_in1, sem_out0, sem_out1):
    wid = lax.axis_index("s") * NC + lax.axis_index("c")
    lanes = lax.iota(jnp.int32, L)
    zero_f = jnp.zeros((L,), jnp.float32)
    one_f = jnp.ones((L,), jnp.float32)
    zero_i = jnp.zeros((L,), jnp.int32)
    ins = (in0, in1)
    outs = (out0, out1)
    bests = (best0, best1)
    sems_in = (sem_in0, sem_in1)
    sems_out = (sem_out0, sem_out1)

    for b in range(NBUF):
        @pl.loop(0, CHUNK)
        def _zero(r, _b=b):
            for j in range(E // L):
                outs[_b][r, pl.ds(j * L, L)] = zero_f

        @pl.loop(0, CHUNK // L)
        def _zero_b(i, _b=b):
            bests[_b][pl.ds(i * L, L)] = zero_i

    row_w = wid * ROWS_W

    def in_copy(s, b):
        return pltpu.make_async_copy(
            in_hbm.at[pl.ds(row_w + s * CHUNK, CHUNK)],
            ins[b],
            sems_in[b])

    def out_copy(s, b):
        return pltpu.make_async_copy(
            outs[b],
            out_hbm.at[pl.ds(row_w + s * CHUNK, CHUNK)],
            sems_out[b])

    in_copy(0, 0).start()
    in_copy(1, 1).start()

    for s in range(NSTEPS):
        b = s % NBUF
        in_copy(s, b).wait()
        if s >= NBUF:
            out_copy(s - NBUF, b).wait()

        @pl.loop(0, GROUPS)
        def _group(g, _b=b):
            rows = g * L + lanes
            oldb = bests[_b][pl.ds(g * L, L)]
            plsc.store_scatter(outs[_b], [rows, oldb], zero_f)
            cc = (lanes + g) & (E - 1)
            m = plsc.load_gather(ins[_b], [rows, cc])
            best = cc
            for c in range(1, E):
                cc = (lanes + (g + c)) & (E - 1)
                v = plsc.load_gather(ins[_b], [rows, cc])
                upd = (v > m) | ((v == m) & (cc < best))
                m = jnp.maximum(m, v)
                best = jnp.where(upd, cc, best)
            plsc.store_scatter(outs[_b], [rows, best], one_f)
            bests[_b][pl.ds(g * L, L)] = best

        out_copy(s, b).start()
        if s + NBUF < NSTEPS:
            in_copy(s + NBUF, b).start()

    out_copy(NSTEPS - 2, (NSTEPS - 2) % NBUF).wait()
    out_copy(NSTEPS - 1, (NSTEPS - 1) % NBUF).wait()


def kernel(snr_probs):
    k = pl.kernel(
        _gating_body,
        out_type=jax.ShapeDtypeStruct((N, E), jnp.float32),
        mesh=plsc.VectorSubcoreMesh(core_axis_name="c", subcore_axis_name="s"),
        compiler_params=pltpu.CompilerParams(
            needs_layout_passes=False,
            use_tc_tiling_on_sc=True,
        ),
        scratch_types=[
            pltpu.VMEM((CHUNK, E), jnp.float32),
            pltpu.VMEM((CHUNK, E), jnp.float32),
            pltpu.VMEM((CHUNK, E), jnp.float32),
            pltpu.VMEM((CHUNK, E), jnp.float32),
            pltpu.VMEM((CHUNK,), jnp.int32),
            pltpu.VMEM((CHUNK,), jnp.int32),
            pltpu.SemaphoreType.DMA,
            pltpu.SemaphoreType.DMA,
            pltpu.SemaphoreType.DMA,
            pltpu.SemaphoreType.DMA,
        ],
    )
    return k(snr_probs)


# speedup vs baseline: 3.5949x; 2.0646x over previous
import jax
import jax.numpy as jnp
from jax import lax
from jax.experimental import pallas as pl
from jax.experimental.pallas import tpu as pltpu
from jax.experimental.pallas import tpu_sc as plsc

E = 64
N = 32768
L = 16
NC, NS = 2, 16
NW = NC * NS
TOKS_W = N // NW
CHUNK = 128
NSTEPS = TOKS_W // CHUNK
GROUPS = CHUNK // L
NBUF = 2


def _gating_body(in_hbm, out_hbm,
                 in0, in1, out0, out1, best0, best1,
                 sem_in0, sem_in1, sem_out0, sem_out1):
    wid = lax.axis_index("s") * NC + lax.axis_index("c")
    lanes = lax.iota(jnp.int32, L)
    zero_f = jnp.zeros((L,), jnp.float32)
    one_f = jnp.ones((L,), jnp.float32)
    zero_i = jnp.zeros((L,), jnp.int32)
    ins = (in0, in1)
    outs = (out0, out1)
    bests = (best0, best1)
    sems_in = (sem_in0, sem_in1)
    sems_out = (sem_out0, sem_out1)

    for b in range(NBUF):
        @pl.loop(0, E)
        def _zero(e, _b=b):
            for j in range(CHUNK // L):
                outs[_b][e, pl.ds(j * L, L)] = zero_f

        @pl.loop(0, CHUNK // L)
        def _zero_b(i, _b=b):
            bests[_b][pl.ds(i * L, L)] = zero_i

    tok_w = wid * TOKS_W

    def in_copy(s, b):
        return pltpu.make_async_copy(
            in_hbm.at[:, pl.ds(tok_w + s * CHUNK, CHUNK)],
            ins[b],
            sems_in[b])

    def out_copy(s, b):
        return pltpu.make_async_copy(
            outs[b],
            out_hbm.at[:, pl.ds(tok_w + s * CHUNK, CHUNK)],
            sems_out[b])

    in_copy(0, 0).start()
    in_copy(1, 1).start()

    for s in range(NSTEPS):
        b = s % NBUF
        in_copy(s, b).wait()
        if s >= NBUF:
            out_copy(s - NBUF, b).wait()

        @pl.loop(0, GROUPS)
        def _group(g, _b=b):
            toks = g * L + lanes
            oldb = bests[_b][pl.ds(g * L, L)]
            plsc.store_scatter(outs[_b], [oldb, toks], zero_f)
            m = ins[_b][0, pl.ds(g * L, L)]
            best = zero_i
            for e in range(1, E):
                v = ins[_b][e, pl.ds(g * L, L)]
                gt = v > m
                m = jnp.maximum(m, v)
                best = jnp.where(gt, jnp.full((L,), e, jnp.int32), best)
            plsc.store_scatter(outs[_b], [best, toks], one_f)
            bests[_b][pl.ds(g * L, L)] = best

        out_copy(s, b).start()
        if s + NBUF < NSTEPS:
            in_copy(s + NBUF, b).start()

    out_copy(NSTEPS - 2, (NSTEPS - 2) % NBUF).wait()
    out_copy(NSTEPS - 1, (NSTEPS - 1) % NBUF).wait()


def kernel(snr_probs):
    k = pl.kernel(
        _gating_body,
        out_type=jax.ShapeDtypeStruct((E, N), jnp.float32),
        mesh=plsc.VectorSubcoreMesh(core_axis_name="c", subcore_axis_name="s"),
        compiler_params=pltpu.CompilerParams(
            needs_layout_passes=False,
            use_tc_tiling_on_sc=True,
        ),
        scratch_types=[
            pltpu.VMEM((E, CHUNK), jnp.float32),
            pltpu.VMEM((E, CHUNK), jnp.float32),
            pltpu.VMEM((E, CHUNK), jnp.float32),
            pltpu.VMEM((E, CHUNK), jnp.float32),
            pltpu.VMEM((CHUNK,), jnp.int32),
            pltpu.VMEM((CHUNK,), jnp.int32),
            pltpu.SemaphoreType.DMA,
            pltpu.SemaphoreType.DMA,
            pltpu.SemaphoreType.DMA,
            pltpu.SemaphoreType.DMA,
        ],
    )
    return k(snr_probs.T).T


# speedup vs baseline: 3.6470x; 1.0145x over previous
import jax
import jax.numpy as jnp
from jax import lax
from jax.experimental import pallas as pl
from jax.experimental.pallas import tpu as pltpu
from jax.experimental.pallas import tpu_sc as plsc

E = 64
N = 32768
L = 16
NC, NS = 2, 16
NW = NC * NS
TOKS_W = N // NW
CHUNK = 256
NSTEPS = TOKS_W // CHUNK
GROUPS = CHUNK // L
NBUF = 2


def _gating_body(in_hbm, out_hbm,
                 in0, in1, out0, out1, best0, best1,
                 sem_in0, sem_in1, sem_out0, sem_out1):
    wid = lax.axis_index("s") * NC + lax.axis_index("c")
    lanes = lax.iota(jnp.int32, L)
    zero_f = jnp.zeros((L,), jnp.float32)
    one_f = jnp.ones((L,), jnp.float32)
    zero_i = jnp.zeros((L,), jnp.int32)
    ins = (in0, in1)
    outs = (out0, out1)
    bests = (best0, best1)
    sems_in = (sem_in0, sem_in1)
    sems_out = (sem_out0, sem_out1)

    for b in range(NBUF):
        @pl.loop(0, E)
        def _zero(e, _b=b):
            for j in range(CHUNK // L):
                outs[_b][e, pl.ds(j * L, L)] = zero_f

        @pl.loop(0, CHUNK // L)
        def _zero_b(i, _b=b):
            bests[_b][pl.ds(i * L, L)] = zero_i

    tok_w = wid * TOKS_W

    def in_copy(s, b):
        return pltpu.make_async_copy(
            in_hbm.at[:, pl.ds(tok_w + s * CHUNK, CHUNK)],
            ins[b],
            sems_in[b])

    def out_copy(s, b):
        return pltpu.make_async_copy(
            outs[b],
            out_hbm.at[:, pl.ds(tok_w + s * CHUNK, CHUNK)],
            sems_out[b])

    in_copy(0, 0).start()
    in_copy(1, 1).start()

    for s in range(NSTEPS):
        b = s % NBUF
        in_copy(s, b).wait()
        if s >= NBUF:
            out_copy(s - NBUF, b).wait()

        @pl.loop(0, GROUPS)
        def _group(g, _b=b):
            toks = g * L + lanes
            oldb = bests[_b][pl.ds(g * L, L)]
            plsc.store_scatter(outs[_b], [oldb, toks], zero_f)
            m = ins[_b][0, pl.ds(g * L, L)]
            best = zero_i
            for e in range(1, E):
                v = ins[_b][e, pl.ds(g * L, L)]
                gt = v > m
                m = jnp.maximum(m, v)
                best = jnp.where(gt, jnp.full((L,), e, jnp.int32), best)
            plsc.store_scatter(outs[_b], [best, toks], one_f)
            bests[_b][pl.ds(g * L, L)] = best

        out_copy(s, b).start()
        if s + NBUF < NSTEPS:
            in_copy(s + NBUF, b).start()

    out_copy(NSTEPS - 2, (NSTEPS - 2) % NBUF).wait()
    out_copy(NSTEPS - 1, (NSTEPS - 1) % NBUF).wait()


def kernel(snr_probs):
    k = pl.kernel(
        _gating_body,
        out_type=jax.ShapeDtypeStruct((E, N), jnp.float32),
        mesh=plsc.VectorSubcoreMesh(core_axis_name="c", subcore_axis_name="s"),
        compiler_params=pltpu.CompilerParams(
            needs_layout_passes=False,
            use_tc_tiling_on_sc=True,
            skip_device_barrier=True,
        ),
        scratch_types=[
            pltpu.VMEM((E, CHUNK), jnp.float32),
            pltpu.VMEM((E, CHUNK), jnp.float32),
            pltpu.VMEM((E, CHUNK), jnp.float32),
            pltpu.VMEM((E, CHUNK), jnp.float32),
            pltpu.VMEM((CHUNK,), jnp.int32),
            pltpu.VMEM((CHUNK,), jnp.int32),
            pltpu.SemaphoreType.DMA,
            pltpu.SemaphoreType.DMA,
            pltpu.SemaphoreType.DMA,
            pltpu.SemaphoreType.DMA,
        ],
    )
    return k(snr_probs.T).T


# speedup vs baseline: 7.9437x; 2.1781x over previous
import jax
import jax.numpy as jnp
from jax import lax
from jax.experimental import pallas as pl
from jax.experimental.pallas import tpu as pltpu

E = 64
N = 32768
BT = 2048


def _gating_block(x_ref, o_ref):
    x = x_ref[...]
    m = jnp.max(x, axis=0, keepdims=True)
    eids = lax.broadcasted_iota(jnp.int32, (E, BT), 0)
    cand = jnp.where(x == m, eids, E)
    idx = jnp.min(cand, axis=0, keepdims=True)
    o_ref[...] = (eids == idx).astype(jnp.float32)


def kernel(snr_probs):
    k = pl.pallas_call(
        _gating_block,
        grid=(N // BT,),
        in_specs=[pl.BlockSpec((E, BT), lambda i: (0, i))],
        out_specs=pl.BlockSpec((E, BT), lambda i: (0, i)),
        out_shape=jax.ShapeDtypeStruct((E, N), jnp.float32),
        compiler_params=pltpu.CompilerParams(
            dimension_semantics=("arbitrary",),
        ),
    )
    return k(snr_probs.T).T
